# baseline (speedup 1.0000x reference)
import jax
import jax.numpy as jnp
from jax import lax
from jax.experimental import pallas as pl
from jax.experimental.pallas import tpu as pltpu
from jax.experimental.pallas import tpu_sc as plsc

N = 10000
E = 320000
H1 = 8
C1 = 8
D1W = H1 * C1
C2 = 40

NPAD = 10240
ROWS_PER_TILE = NPAD // 16
EB = 128
BLK_PER_TILE = 82
NBLK = 32 * BLK_PER_TILE
EPAD = NBLK * EB

SW1 = 80
DW = 32
AW1 = 80
SW2 = 48
AW2 = 48

UNROLL = 8

_GDN = lax.GatherDimensionNumbers(
    offset_dims=(), collapsed_slice_dims=(0,), start_index_map=(0,))


def _vg(v, idx):
    return lax.gather(v, idx[:, None], dimension_numbers=_GDN,
                      slice_sizes=(1,),
                      mode=lax.GatherScatterMode.PROMISE_IN_BOUNDS)


def _lrelu(t):
    return jnp.maximum(t, 0.2 * t)


def _tc1_body(x_ref, w1_ref, asm_ref, adm_ref, s1_ref, d1_ref):
    x = x_ref[...]
    h = jnp.dot(x, w1_ref[...], preferred_element_type=jnp.float32)
    as1 = jnp.dot(h, asm_ref[...], preferred_element_type=jnp.float32)
    ad1 = jnp.dot(h, adm_ref[...], preferred_element_type=jnp.float32)
    s1_ref[...] = jnp.concatenate(
        [h, as1, jnp.zeros((N, 8), jnp.float32)], axis=1)
    gmax = jnp.max(as1, axis=0, keepdims=True)
    m1 = _lrelu(gmax + ad1)
    d1 = jnp.concatenate([ad1, jnp.zeros((N, 8), jnp.float32), m1, m1], axis=1)
    d1_ref[...] = jnp.concatenate(
        [d1, jnp.zeros((NPAD - N, DW), jnp.float32)], axis=0)


def _tc2_body(acc_ref, b1_ref, w2_ref, asm_ref, adm_ref, s2_ref, d2_ref):
    a = acc_ref[...]
    a0 = a[:NPAD]
    a1 = a[NPAD:]
    num = a0[:N, :D1W] + a1[:N, :D1W]
    den = a0[:N, D1W:D1W + H1] + a1[:N, D1W:D1W + H1]
    e8 = (lax.broadcasted_iota(jnp.int32, (H1, D1W), 1) // C1
          == lax.broadcasted_iota(jnp.int32, (H1, D1W), 0)).astype(jnp.float32)
    den_exp = jnp.dot(den, e8, preferred_element_type=jnp.float32)
    o1 = num / (den_exp + 1e-16) + b1_ref[...]
    e1 = jnp.where(o1 > 0, o1, jnp.exp(jnp.minimum(o1, 0.0)) - 1.0)
    h2 = jnp.dot(e1, w2_ref[...], preferred_element_type=jnp.float32)
    as2 = jnp.dot(h2, asm_ref[...], preferred_element_type=jnp.float32)
    ad2 = jnp.dot(h2, adm_ref[...], preferred_element_type=jnp.float32)
    s2_ref[...] = jnp.concatenate([h2, as2], axis=1)
    gmax = jnp.max(as2, axis=0, keepdims=True)
    m2 = _lrelu(gmax + ad2)
    d2 = jnp.concatenate([ad2, ad2, m2, m2], axis=1)
    d2_ref[...] = jnp.concatenate(
        [d2, jnp.zeros((NPAD - N, DW), jnp.float32)], axis=0)


def _tc3_body(acc_ref, b2_ref, out_ref):
    a = acc_ref[...]
    a0 = a[:NPAD]
    a1 = a[NPAD:]
    num = a0[:N, :C2] + a1[:N, :C2]
    den = a0[:N, C2:C2 + 1] + a1[:N, C2:C2 + 1]
    o = num / (den + 1e-16) + b2_ref[...]
    m = jnp.max(o, axis=1, keepdims=True)
    s = o - m
    out_ref[...] = s - jnp.log(jnp.sum(jnp.exp(s), axis=1, keepdims=True))


def _cb1(sbuf, dbuf, mbuf):
    lane = lax.broadcasted_iota(jnp.int32, (16,), 0)
    idx_b = [2 * k + (lane >> 3) for k in range(4)]

    @plsc.parallel_loop(0, EB, unroll=UNROLL)
    def e_body(e):
        dv0 = dbuf[e, pl.ds(0, 16)]
        dv1 = dbuf[e, pl.ds(16, 16)]
        sv4 = sbuf[e, pl.ds(64, 16)]
        t = sv4 + dv0
        ev = jnp.exp(_lrelu(t) - dv1)
        for k in range(4):
            mbuf[e, pl.ds(k * 16, 16)] = (
                sbuf[e, pl.ds(k * 16, 16)] * _vg(ev, idx_b[k]))
        mbuf[e, pl.ds(64, 16)] = ev


def _cb2(sbuf, dbuf, mbuf):
    lane = lax.broadcasted_iota(jnp.int32, (16,), 0)
    idx8 = lane * 0 + 8
    is8 = lane == 8

    @plsc.parallel_loop(0, EB, unroll=UNROLL)
    def e_body(e):
        dv0 = dbuf[e, pl.ds(0, 16)]
        dv1 = dbuf[e, pl.ds(16, 16)]
        sv2 = sbuf[e, pl.ds(32, 16)]
        t = _vg(sv2, idx8) + dv0
        ev = jnp.exp(_lrelu(t) - dv1)
        mbuf[e, pl.ds(0, 16)] = sbuf[e, pl.ds(0, 16)] * ev
        mbuf[e, pl.ds(16, 16)] = sbuf[e, pl.ds(16, 16)] * ev
        mbuf[e, pl.ds(32, 16)] = jnp.where(is8, ev, sv2 * ev)


def _sc_body(compute_block, s_hbm, d_hbm, src_hbm, dst_hbm, z_hbm, out_hbm,
             acc, src_all, dst_all,
             sbuf0, dbuf0, mbuf0, sbuf1, dbuf1, mbuf1,
             gs0, gd0, gs1, gd1, ss0, ss1):
    cid = lax.axis_index("c")
    sid = lax.axis_index("s")
    wid = cid * 16 + sid
    r0 = sid * ROWS_PER_TILE
    pltpu.sync_copy(z_hbm.at[pl.ds(r0, ROWS_PER_TILE)],
                    acc.at[pl.ds(r0, ROWS_PER_TILE)])
    pltpu.sync_copy(src_hbm.at[pl.ds(wid * BLK_PER_TILE, BLK_PER_TILE)],
                    src_all)
    pltpu.sync_copy(dst_hbm.at[pl.ds(wid * BLK_PER_TILE, BLK_PER_TILE)],
                    dst_all)
    plsc.subcore_barrier()

    bufs = ((sbuf0, dbuf0, mbuf0, gs0, gd0, ss0),
            (sbuf1, dbuf1, mbuf1, gs1, gd1, ss1))

    def g_start(b, p):
        sb, db, _, gs, gd, _ = bufs[p]
        pltpu.async_copy(s_hbm.at[src_all.at[b]], sb, gs)
        pltpu.async_copy(d_hbm.at[dst_all.at[b]], db, gd)

    def g_wait(b, p):
        sb, db, _, gs, gd, _ = bufs[p]
        pltpu.make_async_copy(s_hbm.at[src_all.at[b]], sb, gs).wait()
        pltpu.make_async_copy(d_hbm.at[dst_all.at[b]], db, gd).wait()

    def s_start(b, p):
        _, _, mb, _, _, ss = bufs[p]
        pltpu.async_copy(mb, acc.at[dst_all.at[b]], ss, add=True)

    def s_wait(b, p):
        _, _, mb, _, _, ss = bufs[p]
        pltpu.make_async_copy(mb, acc.at[dst_all.at[b]], ss).wait()

    g_start(0, 0)

    def outer(i, c):
        b0 = 2 * i
        b1 = b0 + 1
        b2 = b0 + 2
        g_start(b1, 1)
        g_wait(b0, 0)

        @pl.when(i >= 1)
        def _():
            s_wait(b0, 0)

        compute_block(bufs[0][0], bufs[0][1], bufs[0][2])
        s_start(b0, 0)

        @pl.when(b2 < BLK_PER_TILE)
        def _():
            g_start(b2, 0)

        g_wait(b1, 1)

        @pl.when(i >= 1)
        def _():
            s_wait(b1, 1)

        compute_block(bufs[1][0], bufs[1][1], bufs[1][2])
        s_start(b1, 1)
        return c

    lax.fori_loop(0, BLK_PER_TILE // 2, outer, 0)
    s_wait(0, 0)
    s_wait(1, 1)
    plsc.subcore_barrier()
    pltpu.sync_copy(acc.at[pl.ds(r0, ROWS_PER_TILE)],
                    out_hbm.at[pl.ds(cid * NPAD + r0, ROWS_PER_TILE)])


def _sc1_body(*args):
    _sc_body(_cb1, *args)


def _sc2_body(*args):
    _sc_body(_cb2, *args)


def _make_sc(body, sw, aw):
    mesh = plsc.VectorSubcoreMesh(core_axis_name="c", subcore_axis_name="s")
    return pl.kernel(
        body,
        out_type=jax.ShapeDtypeStruct((2 * NPAD, aw), jnp.float32),
        mesh=mesh,
        scratch_types=[
            pltpu.VMEM_SHARED((NPAD, aw), jnp.float32),
            pltpu.VMEM((BLK_PER_TILE, EB), jnp.int32),
            pltpu.VMEM((BLK_PER_TILE, EB), jnp.int32),
            pltpu.VMEM((EB, sw), jnp.float32),
            pltpu.VMEM((EB, DW), jnp.float32),
            pltpu.VMEM((EB, aw), jnp.float32),
            pltpu.VMEM((EB, sw), jnp.float32),
            pltpu.VMEM((EB, DW), jnp.float32),
            pltpu.VMEM((EB, aw), jnp.float32),
            pltpu.SemaphoreType.DMA,
            pltpu.SemaphoreType.DMA,
            pltpu.SemaphoreType.DMA,
            pltpu.SemaphoreType.DMA,
            pltpu.SemaphoreType.DMA,
            pltpu.SemaphoreType.DMA,
        ],
        compiler_params=pltpu.CompilerParams(use_tc_tiling_on_sc=False),
    )


_sc1 = _make_sc(_sc1_body, SW1, AW1)
_sc2 = _make_sc(_sc2_body, SW2, AW2)

_tc1 = pl.pallas_call(
    _tc1_body,
    out_shape=(jax.ShapeDtypeStruct((N, SW1), jnp.float32),
               jax.ShapeDtypeStruct((NPAD, DW), jnp.float32)),
)

_tc2 = pl.pallas_call(
    _tc2_body,
    out_shape=(jax.ShapeDtypeStruct((N, SW2), jnp.float32),
               jax.ShapeDtypeStruct((NPAD, DW), jnp.float32)),
)

_tc3 = pl.pallas_call(
    _tc3_body,
    out_shape=jax.ShapeDtypeStruct((N, C2), jnp.float32),
)


@jax.jit
def kernel(x, edge_index, W1, att_src1, att_dst1, b1, W2, att_src2, att_dst2,
           b2):
    ei = edge_index.astype(jnp.int32)
    loop = jnp.arange(N, dtype=jnp.int32)
    npd = EPAD - E - N
    pad_dst = N + (jnp.arange(npd, dtype=jnp.int32) % (NPAD - N))
    src = jnp.concatenate([ei[0], loop, jnp.zeros((npd,), jnp.int32)])
    dst = jnp.concatenate([ei[1], loop, pad_dst])
    src = src.reshape(NBLK // 32, 32, EB).transpose(1, 0, 2).reshape(NBLK, EB)
    dst = dst.reshape(NBLK // 32, 32, EB).transpose(1, 0, 2).reshape(NBLK, EB)

    blk = (jnp.arange(D1W)[:, None] // C1 == jnp.arange(H1)[None, :])
    asm1 = att_src1.reshape(D1W, 1) * blk
    adm1 = att_dst1.reshape(D1W, 1) * blk
    asm2 = jnp.tile(att_src2.reshape(C2, 1), (1, 8))
    adm2 = jnp.tile(att_dst2.reshape(C2, 1), (1, 8))
    z1 = jnp.zeros((NPAD, AW1), jnp.float32)
    z2 = jnp.zeros((NPAD, AW2), jnp.float32)

    s1, d1 = _tc1(x, W1, asm1, adm1)
    acc1 = _sc1(s1, d1, src, dst, z1)
    s2, d2 = _tc2(acc1, b1.reshape(1, D1W), W2, asm2, adm2)
    acc2 = _sc2(s2, d2, src, dst, z2)
    return _tc3(acc2, b2.reshape(1, C2))


# speedup vs baseline: 1.0310x; 1.0310x over previous
import jax
import jax.numpy as jnp
from jax import lax
from jax.experimental import pallas as pl
from jax.experimental.pallas import tpu as pltpu
from jax.experimental.pallas import tpu_sc as plsc

N = 10000
E = 320000
H1 = 8
C1 = 8
D1W = H1 * C1
C2 = 40

NPAD = 10240
ROWS_PER_TILE = NPAD // 16
EB = 128
BLK_PER_TILE = 82
NBLK = 32 * BLK_PER_TILE
EPAD = NBLK * EB

SW1 = 80
DW = 16
AW1 = 80
SW2 = 48
AW2 = 48

UNROLL = 4

_GDN = lax.GatherDimensionNumbers(
    offset_dims=(), collapsed_slice_dims=(0,), start_index_map=(0,))


def _vg(v, idx):
    return lax.gather(v, idx[:, None], dimension_numbers=_GDN,
                      slice_sizes=(1,),
                      mode=lax.GatherScatterMode.PROMISE_IN_BOUNDS)


def _lrelu(t):
    return jnp.maximum(t, 0.2 * t)


def _tc1_body(x_ref, w1_ref, asm_ref, adm_ref, s1_ref, d1_ref):
    x = x_ref[...]
    h = jnp.dot(x, w1_ref[...], preferred_element_type=jnp.float32)
    as1 = jnp.dot(h, asm_ref[...], preferred_element_type=jnp.float32)
    ad1 = jnp.dot(h, adm_ref[...], preferred_element_type=jnp.float32)
    s1_ref[...] = jnp.concatenate(
        [h, as1, jnp.zeros((N, 8), jnp.float32)], axis=1)
    gmax = jnp.max(as1, axis=0, keepdims=True)
    m1 = _lrelu(gmax + ad1)
    d1 = jnp.concatenate([ad1, m1], axis=1)
    d1_ref[...] = jnp.concatenate(
        [d1, jnp.zeros((NPAD - N, DW), jnp.float32)], axis=0)


def _tc2_body(acc_ref, b1_ref, w2_ref, asm_ref, adm_ref, s2_ref, d2_ref):
    a = acc_ref[...]
    a0 = a[:NPAD]
    a1 = a[NPAD:]
    num = a0[:N, :D1W] + a1[:N, :D1W]
    den = a0[:N, D1W:D1W + H1] + a1[:N, D1W:D1W + H1]
    e8 = (lax.broadcasted_iota(jnp.int32, (H1, D1W), 1) // C1
          == lax.broadcasted_iota(jnp.int32, (H1, D1W), 0)).astype(jnp.float32)
    den_exp = jnp.dot(den, e8, preferred_element_type=jnp.float32)
    o1 = num / (den_exp + 1e-16) + b1_ref[...]
    e1 = jnp.where(o1 > 0, o1, jnp.exp(jnp.minimum(o1, 0.0)) - 1.0)
    h2 = jnp.dot(e1, w2_ref[...], preferred_element_type=jnp.float32)
    as2 = jnp.dot(h2, asm_ref[...], preferred_element_type=jnp.float32)
    ad2 = jnp.dot(h2, adm_ref[...], preferred_element_type=jnp.float32)
    s2_ref[...] = jnp.concatenate([h2, as2], axis=1)
    gmax = jnp.max(as2, axis=0, keepdims=True)
    m2 = _lrelu(gmax + ad2)
    d2 = jnp.concatenate(
        [ad2[:, :1], m2[:, :1], jnp.zeros((N, DW - 2), jnp.float32)], axis=1)
    d2_ref[...] = jnp.concatenate(
        [d2, jnp.zeros((NPAD - N, DW), jnp.float32)], axis=0)


def _tc3_body(acc_ref, b2_ref, out_ref):
    a = acc_ref[...]
    a0 = a[:NPAD]
    a1 = a[NPAD:]
    num = a0[:N, :C2] + a1[:N, :C2]
    den = a0[:N, C2:C2 + 1] + a1[:N, C2:C2 + 1]
    o = num / (den + 1e-16) + b2_ref[...]
    m = jnp.max(o, axis=1, keepdims=True)
    s = o - m
    out_ref[...] = s - jnp.log(jnp.sum(jnp.exp(s), axis=1, keepdims=True))


def _cb1(sbuf, dbuf, mbuf):
    lane = lax.broadcasted_iota(jnp.int32, (16,), 0)
    idx_m = (lane & 7) + 8
    idx_b = [2 * k + (lane >> 3) for k in range(4)]

    @plsc.parallel_loop(0, EB, unroll=UNROLL)
    def e_body(e):
        dv = dbuf[e]
        sv4 = sbuf[e, pl.ds(64, 16)]
        t = sv4 + dv
        ev = jnp.exp(_lrelu(t) - _vg(dv, idx_m))
        for k in range(4):
            mbuf[e, pl.ds(k * 16, 16)] = (
                sbuf[e, pl.ds(k * 16, 16)] * _vg(ev, idx_b[k]))
        mbuf[e, pl.ds(64, 16)] = ev


def _cb2(sbuf, dbuf, mbuf):
    lane = lax.broadcasted_iota(jnp.int32, (16,), 0)
    idx0 = lane * 0
    idx1 = idx0 + 1
    idx8 = idx0 + 8
    is8 = lane == 8

    @plsc.parallel_loop(0, EB, unroll=UNROLL)
    def e_body(e):
        dv = dbuf[e]
        sv2 = sbuf[e, pl.ds(32, 16)]
        t = _vg(sv2, idx8) + _vg(dv, idx0)
        ev = jnp.exp(_lrelu(t) - _vg(dv, idx1))
        mbuf[e, pl.ds(0, 16)] = sbuf[e, pl.ds(0, 16)] * ev
        mbuf[e, pl.ds(16, 16)] = sbuf[e, pl.ds(16, 16)] * ev
        mbuf[e, pl.ds(32, 16)] = jnp.where(is8, ev, sv2 * ev)


def _sc_body(compute_block, s_hbm, d_hbm, src_hbm, dst_hbm, z_hbm, out_hbm,
             acc, src_all, dst_all,
             sbuf0, dbuf0, mbuf0, sbuf1, dbuf1, mbuf1,
             gs0, gd0, gs1, gd1, ss0, ss1):
    cid = lax.axis_index("c")
    sid = lax.axis_index("s")
    wid = cid * 16 + sid
    r0 = sid * ROWS_PER_TILE
    def _zi(c, u):
        pltpu.sync_copy(z_hbm.at[pl.ds(r0 + c * 128, 128)],
                        acc.at[pl.ds(r0 + c * 128, 128)])
        return u

    lax.fori_loop(0, ROWS_PER_TILE // 128, _zi, 0)
    pltpu.sync_copy(src_hbm.at[pl.ds(wid * BLK_PER_TILE, BLK_PER_TILE)],
                    src_all)
    pltpu.sync_copy(dst_hbm.at[pl.ds(wid * BLK_PER_TILE, BLK_PER_TILE)],
                    dst_all)
    plsc.subcore_barrier()

    bufs = ((sbuf0, dbuf0, mbuf0, gs0, gd0, ss0),
            (sbuf1, dbuf1, mbuf1, gs1, gd1, ss1))

    def g_start(b, p):
        sb, db, _, gs, gd, _ = bufs[p]
        pltpu.async_copy(s_hbm.at[src_all.at[b]], sb, gs)
        pltpu.async_copy(d_hbm.at[dst_all.at[b]], db, gd)

    def g_wait(b, p):
        sb, db, _, gs, gd, _ = bufs[p]
        pltpu.make_async_copy(s_hbm.at[src_all.at[b]], sb, gs).wait()
        pltpu.make_async_copy(d_hbm.at[dst_all.at[b]], db, gd).wait()

    def s_start(b, p):
        _, _, mb, _, _, ss = bufs[p]
        pltpu.async_copy(mb, acc.at[dst_all.at[b]], ss, add=True)

    def s_wait(b, p):
        _, _, mb, _, _, ss = bufs[p]
        pltpu.make_async_copy(mb, acc.at[dst_all.at[b]], ss).wait()

    g_start(0, 0)

    def outer(i, c):
        b0 = 2 * i
        b1 = b0 + 1
        b2 = b0 + 2
        g_start(b1, 1)
        g_wait(b0, 0)

        @pl.when(i >= 1)
        def _():
            s_wait(b0, 0)

        compute_block(bufs[0][0], bufs[0][1], bufs[0][2])
        s_start(b0, 0)

        @pl.when(b2 < BLK_PER_TILE)
        def _():
            g_start(b2, 0)

        g_wait(b1, 1)

        @pl.when(i >= 1)
        def _():
            s_wait(b1, 1)

        compute_block(bufs[1][0], bufs[1][1], bufs[1][2])
        s_start(b1, 1)
        return c

    lax.fori_loop(0, BLK_PER_TILE // 2, outer, 0)
    s_wait(0, 0)
    s_wait(1, 1)
    plsc.subcore_barrier()
    def _zo(c, u):
        pltpu.sync_copy(acc.at[pl.ds(r0 + c * 128, 128)],
                        out_hbm.at[pl.ds(cid * NPAD + r0 + c * 128, 128)])
        return u

    lax.fori_loop(0, ROWS_PER_TILE // 128, _zo, 0)


def _sc1_body(*args):
    _sc_body(_cb1, *args)


def _sc2_body(*args):
    _sc_body(_cb2, *args)


def _make_sc(body, sw, aw):
    mesh = plsc.VectorSubcoreMesh(core_axis_name="c", subcore_axis_name="s")
    return pl.kernel(
        body,
        out_type=jax.ShapeDtypeStruct((2 * NPAD, aw), jnp.float32),
        mesh=mesh,
        scratch_types=[
            pltpu.VMEM_SHARED((NPAD, aw), jnp.float32),
            pltpu.VMEM((BLK_PER_TILE, EB), jnp.int32),
            pltpu.VMEM((BLK_PER_TILE, EB), jnp.int32),
            pltpu.VMEM((EB, sw), jnp.float32),
            pltpu.VMEM((EB, DW), jnp.float32),
            pltpu.VMEM((EB, aw), jnp.float32),
            pltpu.VMEM((EB, sw), jnp.float32),
            pltpu.VMEM((EB, DW), jnp.float32),
            pltpu.VMEM((EB, aw), jnp.float32),
        ] + [pltpu.SemaphoreType.DMA] * 6,
        compiler_params=pltpu.CompilerParams(use_tc_tiling_on_sc=False),
    )


_sc1 = _make_sc(_sc1_body, SW1, AW1)
_sc2 = _make_sc(_sc2_body, SW2, AW2)

_tc1 = pl.pallas_call(
    _tc1_body,
    out_shape=(jax.ShapeDtypeStruct((N, SW1), jnp.float32),
               jax.ShapeDtypeStruct((NPAD, DW), jnp.float32)),
)

_tc2 = pl.pallas_call(
    _tc2_body,
    out_shape=(jax.ShapeDtypeStruct((N, SW2), jnp.float32),
               jax.ShapeDtypeStruct((NPAD, DW), jnp.float32)),
)

_tc3 = pl.pallas_call(
    _tc3_body,
    out_shape=jax.ShapeDtypeStruct((N, C2), jnp.float32),
)


@jax.jit
def kernel(x, edge_index, W1, att_src1, att_dst1, b1, W2, att_src2, att_dst2,
           b2):
    ei = edge_index.astype(jnp.int32)
    loop = jnp.arange(N, dtype=jnp.int32)
    npd = EPAD - E - N
    pad_dst = N + (jnp.arange(npd, dtype=jnp.int32) % (NPAD - N))
    src = jnp.concatenate([ei[0], loop, jnp.zeros((npd,), jnp.int32)])
    dst = jnp.concatenate([ei[1], loop, pad_dst])
    src = src.reshape(NBLK // 32, 32, EB).transpose(1, 0, 2).reshape(NBLK, EB)
    dst = dst.reshape(NBLK // 32, 32, EB).transpose(1, 0, 2).reshape(NBLK, EB)

    blk = (jnp.arange(D1W)[:, None] // C1 == jnp.arange(H1)[None, :])
    asm1 = att_src1.reshape(D1W, 1) * blk
    adm1 = att_dst1.reshape(D1W, 1) * blk
    asm2 = jnp.tile(att_src2.reshape(C2, 1), (1, 8))
    adm2 = jnp.tile(att_dst2.reshape(C2, 1), (1, 8))
    z1 = jnp.zeros((NPAD, AW1), jnp.float32)
    z2 = jnp.zeros((NPAD, AW2), jnp.float32)

    s1, d1 = _tc1(x, W1, asm1, adm1)
    acc1 = _sc1(s1, d1, src, dst, z1)
    s2, d2 = _tc2(acc1, b1.reshape(1, D1W), W2, asm2, adm2)
    acc2 = _sc2(s2, d2, src, dst, z2)
    return _tc3(acc2, b2.reshape(1, C2))
